# baseline (speedup 1.0000x reference)
import functools

import jax
import jax.numpy as jnp
from jax import lax
from jax.experimental import pallas as pl
from jax.experimental.pallas import tpu as pltpu
from jax.experimental.pallas import tpu_sc as plsc

_L = 16


def _permute_cols_sc(x_flat, perm_i32, n_rows, n_cols):
    info = plsc.get_sparse_core_info()
    num_cores, num_subcores = info.num_cores, info.num_subcores
    n_workers = num_cores * num_subcores
    rows_per_w = n_rows // n_workers
    chunk_rows = 4
    n_chunks = rows_per_w // chunk_rows
    chunk_elems = chunk_rows * n_cols

    mesh = plsc.VectorSubcoreMesh(core_axis_name="c", subcore_axis_name="s")

    @functools.partial(
        pl.kernel,
        out_type=jax.ShapeDtypeStruct((n_rows * n_cols,), jnp.float32),
        mesh=mesh,
        scratch_types=[
            pltpu.VMEM((n_cols,), jnp.int32),
            pltpu.VMEM((chunk_elems,), jnp.float32),
            pltpu.VMEM((chunk_elems,), jnp.float32),
            pltpu.VMEM((chunk_elems,), jnp.float32),
            pltpu.VMEM((chunk_elems,), jnp.float32),
            pltpu.VMEM_SHARED((2, num_subcores, chunk_elems), jnp.float32),
            pltpu.VMEM_SHARED((2, num_subcores, chunk_elems), jnp.float32),
            pltpu.SemaphoreType.DMA,
            pltpu.SemaphoreType.DMA,
            pltpu.SemaphoreType.DMA,
            pltpu.SemaphoreType.DMA,
            pltpu.SemaphoreType.DMA,
            pltpu.SemaphoreType.DMA,
            pltpu.SemaphoreType.DMA,
            pltpu.SemaphoreType.DMA,
        ],
        compiler_params=pltpu.CompilerParams(needs_layout_passes=False),
    )
    def k(x_hbm, perm_hbm, out_hbm, perm_v, ti0, ti1, to0, to1, spin, spout,
          si0, si1, mi0, mi1, mo0, mo1, so0, so1):
        s = lax.axis_index("s")
        wid = s * num_cores + lax.axis_index("c")
        base = wid * rows_per_w * n_cols
        tsin = (ti0, ti1)
        tsout = (to0, to1)
        sem_si = (si0, si1)
        sem_mi = (mi0, mi1)
        sem_mo = (mo0, mo1)
        sem_so = (so0, so1)
        pltpu.sync_copy(perm_hbm, perm_v)

        def hslice(ref, g):
            return ref.at[pl.ds(base + g * chunk_elems, chunk_elems)]

        def start_si(g, b):
            pltpu.async_copy(hslice(x_hbm, g), spin.at[b, s], sem_si[b])

        def start_mi(b):
            pltpu.async_copy(spin.at[b, s], tsin[b], sem_mi[b])

        def start_mo(b):
            pltpu.async_copy(tsout[b], spout.at[b, s], sem_mo[b])

        def start_so(g, b):
            pltpu.async_copy(spout.at[b, s], hslice(out_hbm, g), sem_so[b])

        def wait_si(b):
            pltpu.make_async_copy(hslice(x_hbm, 0), spin.at[b, s], sem_si[b]).wait()

        def wait_mi(b):
            pltpu.make_async_copy(spin.at[b, s], tsin[b], sem_mi[b]).wait()

        def wait_mo(b):
            pltpu.make_async_copy(tsout[b], spout.at[b, s], sem_mo[b]).wait()

        def wait_so(b):
            pltpu.make_async_copy(spout.at[b, s], hslice(out_hbm, 0), sem_so[b]).wait()

        def compute(b):
            @plsc.parallel_loop(0, n_cols, step=_L, unroll=8)
            def col_body(cbase):
                col = perm_v[pl.ds(cbase, _L)]
                for r in range(chunk_rows):
                    val = plsc.load_gather(tsin[b], [col + r * n_cols])
                    tsout[b][pl.ds(r * n_cols + cbase, _L)] = val

        def tick(t, par, guard):
            if not guard or 0 <= t - 2 < n_chunks:
                wait_mi(par)
            if not guard or t < n_chunks:
                start_si(t, par)
            if not guard or 0 <= t - 1 < n_chunks:
                wait_si(1 - par)
                start_mi(1 - par)
            if not guard or 0 <= t - 4 < n_chunks:
                wait_so(par)
            if not guard or 0 <= t - 2 < n_chunks:
                compute(par)
                start_mo(par)
            if not guard or 0 <= t - 3 < n_chunks:
                wait_mo(1 - par)
                start_so(t - 3, 1 - par)

        for t in range(4):
            tick(t, t % 2, True)

        def steady(i, carry):
            t = 4 + 2 * i
            tick(t, 0, False)
            tick(t + 1, 1, False)
            return carry

        lax.fori_loop(0, (n_chunks - 4) // 2, steady, 0, unroll=1)
        for t in range(n_chunks, n_chunks + 4):
            tick(t, t % 2, True)

    return k(x_flat, perm_i32)


def kernel(x, perm):
    n_rows, n_cols = x.shape
    out_flat = _permute_cols_sc(
        x.reshape(n_rows * n_cols), perm.astype(jnp.int32), n_rows, n_cols
    )
    return out_flat.reshape(n_rows, n_cols)
